# Initial kernel scaffold; baseline (speedup 1.0000x reference)
import jax
import jax.numpy as jnp
from jax.experimental import pallas as pl


def kernel(z, edge_index, edge_attr, W1, b1, W2, b2):
    raise NotImplementedError("write your pallas kernel here")


# speedup vs baseline: 5.4609x; 5.4609x over previous
import functools

import jax
import jax.numpy as jnp
from jax import lax
from jax.experimental import pallas as pl
from jax.experimental.pallas import tpu as pltpu
from jax.experimental.pallas import tpu_sc as plsc

N = 10000
E = 320000
D = 128

NC = 2
NS = 16
NW = NC * NS

NPAD = 10240
NBLK = NPAD // 128
EPAD = 327680
CHUNK = 128
NCHUNK = EPAD // (NW * CHUNK)
ROWS_PER_TILE = NPAD // NS

_mesh = plsc.VectorSubcoreMesh(core_axis_name="c", subcore_axis_name="s")


def _zero_vmem_block(ref, nrows):
    def body(k, _):
        for dd in range(8):
            ref[k, pl.ds(dd * 16, 16)] = jnp.zeros((16,), jnp.float32)
        return 0
    lax.fori_loop(0, nrows, body, 0)


def _deg_body(dstp_hbm, ewp_hbm, deg_out_hbm, dst_v, ew_v, z_v, dacc, sem):
    c = lax.axis_index("c")
    s = lax.axis_index("s")
    wid = c * NS + s

    def zb(i, _):
        z_v[pl.ds(i * 16, 16)] = jnp.zeros((16,), jnp.float32)
        return 0
    lax.fori_loop(0, ROWS_PER_TILE // 16, zb, 0)
    pltpu.sync_copy(z_v, dacc.at[pl.ds(s * ROWS_PER_TILE, ROWS_PER_TILE)])
    plsc.subcore_barrier()

    def chunk(j, _):
        pltpu.sync_copy(dstp_hbm.at[wid, j], dst_v)
        pltpu.sync_copy(ewp_hbm.at[wid, j], ew_v)
        pltpu.sync_copy(ew_v, dacc.at[dst_v], add=True)
        return 0
    lax.fori_loop(0, NCHUNK, chunk, 0)
    plsc.subcore_barrier()
    pltpu.sync_copy(dacc.at[pl.ds(s * ROWS_PER_TILE, ROWS_PER_TILE)],
                    deg_out_hbm.at[c, pl.ds(s * ROWS_PER_TILE, ROWS_PER_TILE)])


_deg_kernel = pl.kernel(
    _deg_body,
    out_type=jax.ShapeDtypeStruct((NC, NPAD), jnp.float32),
    mesh=_mesh,
    scratch_types=[
        pltpu.VMEM((CHUNK,), jnp.int32),
        pltpu.VMEM((CHUNK,), jnp.float32),
        pltpu.VMEM((ROWS_PER_TILE,), jnp.float32),
        pltpu.VMEM_SHARED((NPAD,), jnp.float32),
        pltpu.SemaphoreType.DMA,
    ],
)


def _edge_body(g_hbm, srcp_hbm, dstp_hbm, ewp_hbm, out_hbm,
               src_v, dst_v, ew_v, rows_v, zrow_v, acc, sem):
    c = lax.axis_index("c")
    s = lax.axis_index("s")
    wid = c * NS + s

    _zero_vmem_block(zrow_v, 128)
    for m in range(ROWS_PER_TILE // 128):
        pltpu.sync_copy(zrow_v, acc.at[pl.ds(s * ROWS_PER_TILE + m * 128, 128)])
    plsc.subcore_barrier()

    def chunk(j, _):
        pltpu.sync_copy(srcp_hbm.at[wid, j], src_v)
        pltpu.sync_copy(dstp_hbm.at[wid, j], dst_v)
        pltpu.sync_copy(ewp_hbm.at[wid, j], ew_v)
        pltpu.async_copy(g_hbm.at[src_v], rows_v, sem).wait()

        def scale(gi, _):
            ew16 = ew_v[pl.ds(gi * 16, 16)]
            for t in range(16):
                w = ew16.at[jnp.full((16,), t, jnp.int32)].get(
                    mode="promise_in_bounds")
                k = gi * 16 + t
                for dd in range(8):
                    sl = pl.ds(dd * 16, 16)
                    rows_v[k, sl] = rows_v[k, sl] * w
            return 0
        lax.fori_loop(0, CHUNK // 16, scale, 0)
        pltpu.sync_copy(rows_v, acc.at[dst_v], add=True)
        return 0
    lax.fori_loop(0, NCHUNK, chunk, 0)
    plsc.subcore_barrier()
    for m in range(ROWS_PER_TILE // 128):
        pltpu.sync_copy(acc.at[pl.ds(s * ROWS_PER_TILE + m * 128, 128)],
                        out_hbm.at[c, pl.ds(s * ROWS_PER_TILE + m * 128, 128)])


_edge_kernel = pl.kernel(
    _edge_body,
    out_type=jax.ShapeDtypeStruct((NC, NPAD, D), jnp.float32),
    mesh=_mesh,
    scratch_types=[
        pltpu.VMEM((CHUNK,), jnp.int32),
        pltpu.VMEM((CHUNK,), jnp.int32),
        pltpu.VMEM((CHUNK,), jnp.float32),
        pltpu.VMEM((CHUNK, D), jnp.float32),
        pltpu.VMEM((128, D), jnp.float32),
        pltpu.VMEM_SHARED((NPAD, D), jnp.float32),
        pltpu.SemaphoreType.DMA,
    ],
)


def _diag(vec_1x128):
    r = lax.broadcasted_iota(jnp.int32, (128, 128), 0)
    cidx = lax.broadcasted_iota(jnp.int32, (128, 128), 1)
    b = jnp.broadcast_to(vec_1x128, (128, 128))
    return jnp.where(r == cidx, b, jnp.zeros((128, 128), jnp.float32))


def _tc_pre_body(deg_ref, z_ref, w_ref, dinv_ref, g_ref):
    deg = deg_ref[0:1, :] + deg_ref[1:2, :] + 1.0
    dinv = lax.rsqrt(deg)
    dinv_ref[...] = dinv.reshape(128)
    dmat = _diag(dinv)
    zh = jnp.dot(z_ref[...], w_ref[...], preferred_element_type=jnp.float32)
    g_ref[...] = jnp.dot(dmat, zh, preferred_element_type=jnp.float32)


def _tc_mid_body(p_ref, g_ref, dinv_ref, b_ref, w_ref, g2_ref):
    dmat = _diag(dinv_ref[...].reshape(1, 128))
    t = p_ref[0] + p_ref[1] + g_ref[...]
    x = jnp.dot(dmat, t, preferred_element_type=jnp.float32) + b_ref[...]
    x = jnp.maximum(x, 0.0)
    xh = jnp.dot(x, w_ref[...], preferred_element_type=jnp.float32)
    g2_ref[...] = jnp.dot(dmat, xh, preferred_element_type=jnp.float32)


def _tc_post_body(q_ref, g_ref, dinv_ref, b_ref, out_ref):
    dmat = _diag(dinv_ref[...].reshape(1, 128))
    t = q_ref[0] + q_ref[1] + g_ref[...]
    out_ref[...] = jnp.dot(dmat, t, preferred_element_type=jnp.float32) + b_ref[...]


_tc_pre = pl.pallas_call(
    _tc_pre_body,
    grid=(NBLK,),
    in_specs=[
        pl.BlockSpec((NC, 128), lambda i: (0, i)),
        pl.BlockSpec((128, D), lambda i: (i, 0)),
        pl.BlockSpec((D, D), lambda i: (0, 0)),
    ],
    out_specs=[
        pl.BlockSpec((128,), lambda i: (i,)),
        pl.BlockSpec((128, D), lambda i: (i, 0)),
    ],
    out_shape=[
        jax.ShapeDtypeStruct((NPAD,), jnp.float32),
        jax.ShapeDtypeStruct((NPAD, D), jnp.float32),
    ],
)

_tc_mid = pl.pallas_call(
    _tc_mid_body,
    grid=(NBLK,),
    in_specs=[
        pl.BlockSpec((NC, 128, D), lambda i: (0, i, 0)),
        pl.BlockSpec((128, D), lambda i: (i, 0)),
        pl.BlockSpec((128,), lambda i: (i,)),
        pl.BlockSpec((1, D), lambda i: (0, 0)),
        pl.BlockSpec((D, D), lambda i: (0, 0)),
    ],
    out_specs=pl.BlockSpec((128, D), lambda i: (i, 0)),
    out_shape=jax.ShapeDtypeStruct((NPAD, D), jnp.float32),
)

_tc_post = pl.pallas_call(
    _tc_post_body,
    grid=(NBLK,),
    in_specs=[
        pl.BlockSpec((NC, 128, D), lambda i: (0, i, 0)),
        pl.BlockSpec((128, D), lambda i: (i, 0)),
        pl.BlockSpec((128,), lambda i: (i,)),
        pl.BlockSpec((1, D), lambda i: (0, 0)),
    ],
    out_specs=pl.BlockSpec((128, D), lambda i: (i, 0)),
    out_shape=jax.ShapeDtypeStruct((NPAD, D), jnp.float32),
)


@jax.jit
def kernel(z, edge_index, edge_attr, W1, b1, W2, b2):
    src = edge_index[0].astype(jnp.int32)
    dst = edge_index[1].astype(jnp.int32)
    ew = edge_attr.astype(jnp.float32)

    pad = EPAD - E
    srcp = jnp.concatenate([src, jnp.zeros((pad,), jnp.int32)]).reshape(NW, NCHUNK, CHUNK)
    dstp = jnp.concatenate([dst, jnp.full((pad,), NPAD - 1, jnp.int32)]).reshape(NW, NCHUNK, CHUNK)
    ewp = jnp.concatenate([ew, jnp.zeros((pad,), jnp.float32)]).reshape(NW, NCHUNK, CHUNK)

    zp = jnp.pad(z, ((0, NPAD - N), (0, 0)))
    b1r = b1.reshape(1, D)
    b2r = b2.reshape(1, D)

    degp = _deg_kernel(dstp, ewp)
    dinv, g1 = _tc_pre(degp, zp, W1)
    p = _edge_kernel(g1, srcp, dstp, ewp)
    g2 = _tc_mid(p, g1, dinv, b1r, W2)
    q = _edge_kernel(g2, srcp, dstp, ewp)
    out = _tc_post(q, g2, dinv, b2r)
    return out[:N]


# speedup vs baseline: 7.2985x; 1.3365x over previous
import functools

import jax
import jax.numpy as jnp
from jax import lax
from jax.experimental import pallas as pl
from jax.experimental.pallas import tpu as pltpu
from jax.experimental.pallas import tpu_sc as plsc

N = 10000
E = 320000
D = 128

NC = 2
NS = 16
NW = NC * NS

NPAD = 10240
NBLK = NPAD // 128
EPAD = 327680
CHUNK = 128
NCHUNK = EPAD // (NW * CHUNK)
ROWS_PER_TILE = NPAD // NS

_mesh = plsc.VectorSubcoreMesh(core_axis_name="c", subcore_axis_name="s")


def _zero_vmem_block(ref, nrows):
    def body(k, _):
        for dd in range(8):
            ref[k, pl.ds(dd * 16, 16)] = jnp.zeros((16,), jnp.float32)
        return 0
    lax.fori_loop(0, nrows, body, 0)


def _deg_body(dstp_hbm, ewp_hbm, deg_out_hbm, dst_v, ew_v, z_v, dacc, sem):
    c = lax.axis_index("c")
    s = lax.axis_index("s")
    wid = c * NS + s

    def zb(i, _):
        z_v[pl.ds(i * 16, 16)] = jnp.zeros((16,), jnp.float32)
        return 0
    lax.fori_loop(0, ROWS_PER_TILE // 16, zb, 0)
    pltpu.sync_copy(z_v, dacc.at[pl.ds(s * ROWS_PER_TILE, ROWS_PER_TILE)])
    plsc.subcore_barrier()

    def chunk(j, _):
        pltpu.sync_copy(dstp_hbm.at[wid, j], dst_v)
        pltpu.sync_copy(ewp_hbm.at[wid, j], ew_v)
        pltpu.sync_copy(ew_v, dacc.at[dst_v], add=True)
        return 0
    lax.fori_loop(0, NCHUNK, chunk, 0)
    plsc.subcore_barrier()
    pltpu.sync_copy(dacc.at[pl.ds(s * ROWS_PER_TILE, ROWS_PER_TILE)],
                    deg_out_hbm.at[c, pl.ds(s * ROWS_PER_TILE, ROWS_PER_TILE)])


_deg_kernel = pl.kernel(
    _deg_body,
    out_type=jax.ShapeDtypeStruct((NC, NPAD), jnp.float32),
    mesh=_mesh,
    scratch_types=[
        pltpu.VMEM((CHUNK,), jnp.int32),
        pltpu.VMEM((CHUNK,), jnp.float32),
        pltpu.VMEM((ROWS_PER_TILE,), jnp.float32),
        pltpu.VMEM_SHARED((NPAD,), jnp.float32),
        pltpu.SemaphoreType.DMA,
    ],
)


def _edge_body(g_hbm, pack_hbm, ewp_hbm, out_hbm,
               ib0, ib1, eb0, eb1, rows0, rows1, acc,
               is0, is1, gs0, gs1, ss0, ss1):
    c = lax.axis_index("c")
    s = lax.axis_index("s")
    wid = c * NS + s
    ib = (ib0, ib1)
    eb = (eb0, eb1)
    rows = (rows0, rows1)
    isem = (is0, is1)
    gsem = (gs0, gs1)
    ssem = (ss0, ss1)

    _zero_vmem_block(rows0, 128)
    for m in range(ROWS_PER_TILE // 128):
        pltpu.sync_copy(rows0, acc.at[pl.ds(s * ROWS_PER_TILE + m * 128, 128)])
    plsc.subcore_barrier()

    def scale(buf, ebuf):
        def body(gi, _):
            ew16 = ebuf[pl.ds(gi * 16, 16)]
            for t in range(16):
                w = ew16.at[jnp.full((16,), t, jnp.int32)].get(
                    mode="promise_in_bounds")
                k = gi * 16 + t
                for dd in range(8):
                    sl = pl.ds(dd * 16, 16)
                    buf[k, sl] = buf[k, sl] * w
            return 0
        lax.fori_loop(0, CHUNK // 16, body, 0)

    pltpu.sync_copy(pack_hbm.at[wid, 0], ib0)
    pltpu.sync_copy(ewp_hbm.at[wid, 0], eb0)
    pltpu.async_copy(g_hbm.at[ib0.at[0]], rows0, gs0)

    def pair(i, _):
        for p in (0, 1):
            j = 2 * i + p
            q = 1 - p
            have_prev = (i > 0) if p == 0 else True
            have_next = True if p == 0 else (i < NCHUNK // 2 - 1)

            def maybe(cond, fn):
                if cond is True:
                    fn()
                else:
                    pl.when(cond)(fn)

            pltpu.make_async_copy(g_hbm.at[ib[p].at[0]], rows[p],
                                  gsem[p]).wait()
            def wait_prev():
                pltpu.make_async_copy(rows[q], acc.at[ib[q].at[1]],
                                      ssem[q]).wait()

            def stage_next():
                pltpu.async_copy(pack_hbm.at[wid, j + 1], ib[q], isem[q])
                pltpu.async_copy(ewp_hbm.at[wid, j + 1], eb[q], isem[q])
                return None
            maybe(have_prev, wait_prev)
            maybe(have_next, stage_next)
            scale(rows[p], eb[p])

            def start_next():
                pltpu.make_async_copy(pack_hbm.at[wid, 0], ib[q],
                                      isem[q]).wait()
                pltpu.make_async_copy(ewp_hbm.at[wid, 0], eb[q],
                                      isem[q]).wait()
                pltpu.async_copy(g_hbm.at[ib[q].at[0]], rows[q], gsem[q])
            maybe(have_next, start_next)
            pltpu.async_copy(rows[p], acc.at[ib[p].at[1]], ssem[p], add=True)
        return 0
    lax.fori_loop(0, NCHUNK // 2, pair, 0)
    pltpu.make_async_copy(rows1, acc.at[ib1.at[1]], ssem[1]).wait()
    plsc.subcore_barrier()
    for m in range(ROWS_PER_TILE // 128):
        pltpu.sync_copy(acc.at[pl.ds(s * ROWS_PER_TILE + m * 128, 128)],
                        out_hbm.at[c, pl.ds(s * ROWS_PER_TILE + m * 128, 128)])


_edge_kernel = pl.kernel(
    _edge_body,
    out_type=jax.ShapeDtypeStruct((NC, NPAD, D), jnp.float32),
    mesh=_mesh,
    scratch_types=[
        pltpu.VMEM((2, CHUNK), jnp.int32),
        pltpu.VMEM((2, CHUNK), jnp.int32),
        pltpu.VMEM((CHUNK,), jnp.float32),
        pltpu.VMEM((CHUNK,), jnp.float32),
        pltpu.VMEM((CHUNK, D), jnp.float32),
        pltpu.VMEM((CHUNK, D), jnp.float32),
        pltpu.VMEM_SHARED((NPAD, D), jnp.float32),
        pltpu.SemaphoreType.DMA,
        pltpu.SemaphoreType.DMA,
        pltpu.SemaphoreType.DMA,
        pltpu.SemaphoreType.DMA,
        pltpu.SemaphoreType.DMA,
        pltpu.SemaphoreType.DMA,
    ],
)


def _diag(vec_1x128):
    r = lax.broadcasted_iota(jnp.int32, (128, 128), 0)
    cidx = lax.broadcasted_iota(jnp.int32, (128, 128), 1)
    b = jnp.broadcast_to(vec_1x128, (128, 128))
    return jnp.where(r == cidx, b, jnp.zeros((128, 128), jnp.float32))


def _tc_pre_body(deg_ref, z_ref, w_ref, dinv_ref, g_ref):
    deg = deg_ref[0:1, :] + deg_ref[1:2, :] + 1.0
    dinv = lax.rsqrt(deg)
    dinv_ref[...] = dinv.reshape(128)
    dmat = _diag(dinv)
    zh = jnp.dot(z_ref[...], w_ref[...], preferred_element_type=jnp.float32)
    g_ref[...] = jnp.dot(dmat, zh, preferred_element_type=jnp.float32)


def _tc_mid_body(p_ref, g_ref, dinv_ref, b_ref, w_ref, g2_ref):
    dmat = _diag(dinv_ref[...].reshape(1, 128))
    t = p_ref[0] + p_ref[1] + g_ref[...]
    x = jnp.dot(dmat, t, preferred_element_type=jnp.float32) + b_ref[...]
    x = jnp.maximum(x, 0.0)
    xh = jnp.dot(x, w_ref[...], preferred_element_type=jnp.float32)
    g2_ref[...] = jnp.dot(dmat, xh, preferred_element_type=jnp.float32)


def _tc_post_body(q_ref, g_ref, dinv_ref, b_ref, out_ref):
    dmat = _diag(dinv_ref[...].reshape(1, 128))
    t = q_ref[0] + q_ref[1] + g_ref[...]
    out_ref[...] = jnp.dot(dmat, t, preferred_element_type=jnp.float32) + b_ref[...]


_tc_pre = pl.pallas_call(
    _tc_pre_body,
    grid=(NBLK,),
    in_specs=[
        pl.BlockSpec((NC, 128), lambda i: (0, i)),
        pl.BlockSpec((128, D), lambda i: (i, 0)),
        pl.BlockSpec((D, D), lambda i: (0, 0)),
    ],
    out_specs=[
        pl.BlockSpec((128,), lambda i: (i,)),
        pl.BlockSpec((128, D), lambda i: (i, 0)),
    ],
    out_shape=[
        jax.ShapeDtypeStruct((NPAD,), jnp.float32),
        jax.ShapeDtypeStruct((NPAD, D), jnp.float32),
    ],
)

_tc_mid = pl.pallas_call(
    _tc_mid_body,
    grid=(NBLK,),
    in_specs=[
        pl.BlockSpec((NC, 128, D), lambda i: (0, i, 0)),
        pl.BlockSpec((128, D), lambda i: (i, 0)),
        pl.BlockSpec((128,), lambda i: (i,)),
        pl.BlockSpec((1, D), lambda i: (0, 0)),
        pl.BlockSpec((D, D), lambda i: (0, 0)),
    ],
    out_specs=pl.BlockSpec((128, D), lambda i: (i, 0)),
    out_shape=jax.ShapeDtypeStruct((NPAD, D), jnp.float32),
)

_tc_post = pl.pallas_call(
    _tc_post_body,
    grid=(NBLK,),
    in_specs=[
        pl.BlockSpec((NC, 128, D), lambda i: (0, i, 0)),
        pl.BlockSpec((128, D), lambda i: (i, 0)),
        pl.BlockSpec((128,), lambda i: (i,)),
        pl.BlockSpec((1, D), lambda i: (0, 0)),
    ],
    out_specs=pl.BlockSpec((128, D), lambda i: (i, 0)),
    out_shape=jax.ShapeDtypeStruct((NPAD, D), jnp.float32),
)


@jax.jit
def kernel(z, edge_index, edge_attr, W1, b1, W2, b2):
    src = edge_index[0].astype(jnp.int32)
    dst = edge_index[1].astype(jnp.int32)
    ew = edge_attr.astype(jnp.float32)

    pad = EPAD - E
    srcp = jnp.concatenate([src, jnp.zeros((pad,), jnp.int32)]).reshape(NW, NCHUNK, CHUNK)
    dstp = jnp.concatenate([dst, jnp.full((pad,), NPAD - 1, jnp.int32)]).reshape(NW, NCHUNK, CHUNK)
    ewp = jnp.concatenate([ew, jnp.zeros((pad,), jnp.float32)]).reshape(NW, NCHUNK, CHUNK)
    packp = jnp.stack([srcp, dstp], axis=2)

    zp = jnp.pad(z, ((0, NPAD - N), (0, 0)))
    b1r = b1.reshape(1, D)
    b2r = b2.reshape(1, D)

    degp = _deg_kernel(dstp, ewp)
    dinv, g1 = _tc_pre(degp, zp, W1)
    p = _edge_kernel(g1, packp, ewp)
    g2 = _tc_mid(p, g1, dinv, b1r, W2)
    q = _edge_kernel(g2, packp, ewp)
    out = _tc_post(q, g2, dinv, b2r)
    return out[:N]
